# Initial kernel scaffold; baseline (speedup 1.0000x reference)
import jax
import jax.numpy as jnp
from jax import lax
from jax.experimental import pallas as pl
from jax.experimental.pallas import tpu as pltpu
from jax.experimental.pallas import tpu_sc as plsc

N = 10000
E = 320000
D = 128
DE = 16

NB = 1024
EB = 4000

NC = 2
NS = 16
NT = NC * NS
EPT = E // NT
CH = 80
NCHUNK = EPT // CH
NPT = N // NS


def _prep_body(x_ref, w_ref, asrc_ref, adst_ref, h_ref, ss_ref, sd_ref):
    x = x_ref[...]
    nrm = jnp.sqrt(jnp.sum(x * x, axis=1, keepdims=True))
    nrm = jnp.where(nrm == 0.0, 1e-8, nrm)
    h = jnp.dot(x / nrm, w_ref[...], preferred_element_type=jnp.float32)
    h_ref[...] = h
    ss_ref[...] = jnp.sum(h * asrc_ref[...], axis=1, keepdims=True)
    sd_ref[...] = jnp.sum(h * adst_ref[...], axis=1, keepdims=True)


def _prep(x, W, a_src, a_dst):
    grid = (N + NB - 1) // NB
    return pl.pallas_call(
        _prep_body,
        grid=(grid,),
        in_specs=[
            pl.BlockSpec((NB, D), lambda i: (i, 0)),
            pl.BlockSpec((D, D), lambda i: (0, 0)),
            pl.BlockSpec((1, D), lambda i: (0, 0)),
            pl.BlockSpec((1, D), lambda i: (0, 0)),
        ],
        out_specs=[
            pl.BlockSpec((NB, D), lambda i: (i, 0)),
            pl.BlockSpec((NB, 1), lambda i: (i, 0)),
            pl.BlockSpec((NB, 1), lambda i: (i, 0)),
        ],
        out_shape=[
            jax.ShapeDtypeStruct((N, D), jnp.float32),
            jax.ShapeDtypeStruct((N, 1), jnp.float32),
            jax.ShapeDtypeStruct((N, 1), jnp.float32),
        ],
    )(x, W, a_src[None, :], a_dst[None, :])


def _ea_body(ea_ref, we1_ref, ae1_ref, we2_ref, ae2_ref, we3_ref, ae3_ref,
             o1_ref, o2_ref, o3_ref, easum_ref):
    ea = ea_ref[...]
    for we_ref, ae_ref, o_ref in ((we1_ref, ae1_ref, o1_ref),
                                  (we2_ref, ae2_ref, o2_ref),
                                  (we3_ref, ae3_ref, o3_ref)):
        v = jnp.sum(we_ref[...] * ae_ref[...], axis=1)
        o_ref[...] = jnp.sum(ea * v[None, :], axis=1, keepdims=True)

    @pl.when(pl.program_id(0) == 0)
    def _():
        easum_ref[...] = jnp.zeros_like(easum_ref)

    easum_ref[...] += jnp.sum(ea, axis=0, keepdims=True)


def _edge_scalars(edge_attr, We1, a_e1, We2, a_e2, We3, a_e3):
    grid = E // EB
    return pl.pallas_call(
        _ea_body,
        grid=(grid,),
        in_specs=[
            pl.BlockSpec((EB, DE), lambda i: (i, 0)),
            pl.BlockSpec((DE, D), lambda i: (0, 0)),
            pl.BlockSpec((1, D), lambda i: (0, 0)),
            pl.BlockSpec((DE, D), lambda i: (0, 0)),
            pl.BlockSpec((1, D), lambda i: (0, 0)),
            pl.BlockSpec((DE, D), lambda i: (0, 0)),
            pl.BlockSpec((1, D), lambda i: (0, 0)),
        ],
        out_specs=[
            pl.BlockSpec((EB, 1), lambda i: (i, 0)),
            pl.BlockSpec((EB, 1), lambda i: (i, 0)),
            pl.BlockSpec((EB, 1), lambda i: (i, 0)),
            pl.BlockSpec((1, DE), lambda i: (0, 0)),
        ],
        out_shape=[
            jax.ShapeDtypeStruct((E, 1), jnp.float32),
            jax.ShapeDtypeStruct((E, 1), jnp.float32),
            jax.ShapeDtypeStruct((E, 1), jnp.float32),
            jax.ShapeDtypeStruct((1, DE), jnp.float32),
        ],
    )(edge_attr, We1, a_e1[None, :], We2, a_e2[None, :], We3, a_e3[None, :])


def _gelu(o):
    return 0.5 * o * (1.0 + lax.erf(o * (2.0 ** -0.5)))


def _loop_ex(ss, sd, easum_ref, we_ref, ae_ref):
    v = jnp.sum(we_ref[...] * ae_ref[...], axis=1)
    loop_ae = jnp.sum(easum_ref[...] * v[None, :]) / E
    s = ss + sd + loop_ae
    s = jnp.where(s >= 0.0, s, 0.2 * s)
    return jnp.exp(s)


def _fin_mid_body(acc_ref, den_ref, h_ref, ss_ref, sd_ref, b_ref,
                  easum_ref, we_ref, ae_ref,
                  wn_ref, asrcn_ref, adstn_ref,
                  hn_ref, ssn_ref, sdn_ref):
    exl = _loop_ex(ss_ref[..., 0], sd_ref[..., 0], easum_ref, we_ref, ae_ref)
    h = h_ref[...]
    acc = acc_ref[0] + acc_ref[1] + exl[:, None] * h
    den = jnp.sum(den_ref[...], axis=0) + exl
    g = _gelu(acc / (den[:, None] + 1e-16) + b_ref[...])
    hn = jnp.dot(g, wn_ref[...], preferred_element_type=jnp.float32)
    hn_ref[...] = hn
    ssn_ref[...] = jnp.sum(hn * asrcn_ref[...], axis=1, keepdims=True)
    sdn_ref[...] = jnp.sum(hn * adstn_ref[...], axis=1, keepdims=True)


def _fin_last_body(acc_ref, den_ref, h_ref, ss_ref, sd_ref, b_ref,
                   easum_ref, we_ref, ae_ref, out_ref):
    exl = _loop_ex(ss_ref[..., 0], sd_ref[..., 0], easum_ref, we_ref, ae_ref)
    h = h_ref[...]
    acc = acc_ref[0] + acc_ref[1] + exl[:, None] * h
    den = jnp.sum(den_ref[...], axis=0) + exl
    out_ref[...] = _gelu(acc / (den[:, None] + 1e-16) + b_ref[...])


def _fin_specs():
    return [
        pl.BlockSpec((2, NB, D), lambda i: (0, i, 0)),
        pl.BlockSpec((NT, NB), lambda i: (0, i)),
        pl.BlockSpec((NB, D), lambda i: (i, 0)),
        pl.BlockSpec((NB, 1), lambda i: (i, 0)),
        pl.BlockSpec((NB, 1), lambda i: (i, 0)),
        pl.BlockSpec((1, D), lambda i: (0, 0)),
        pl.BlockSpec((1, DE), lambda i: (0, 0)),
        pl.BlockSpec((DE, D), lambda i: (0, 0)),
        pl.BlockSpec((1, D), lambda i: (0, 0)),
    ]


def _finalize_mid(acc, den, h, ss, sd, b, easum, We, a_e, Wn, asrcn, adstn):
    grid = (N + NB - 1) // NB
    return pl.pallas_call(
        _fin_mid_body,
        grid=(grid,),
        in_specs=_fin_specs() + [
            pl.BlockSpec((D, D), lambda i: (0, 0)),
            pl.BlockSpec((1, D), lambda i: (0, 0)),
            pl.BlockSpec((1, D), lambda i: (0, 0)),
        ],
        out_specs=[
            pl.BlockSpec((NB, D), lambda i: (i, 0)),
            pl.BlockSpec((NB, 1), lambda i: (i, 0)),
            pl.BlockSpec((NB, 1), lambda i: (i, 0)),
        ],
        out_shape=[
            jax.ShapeDtypeStruct((N, D), jnp.float32),
            jax.ShapeDtypeStruct((N, 1), jnp.float32),
            jax.ShapeDtypeStruct((N, 1), jnp.float32),
        ],
    )(acc, den, h, ss, sd, b[None, :], easum, We, a_e[None, :],
      Wn, asrcn[None, :], adstn[None, :])


def _finalize_last(acc, den, h, ss, sd, b, easum, We, a_e):
    grid = (N + NB - 1) // NB
    return pl.pallas_call(
        _fin_last_body,
        grid=(grid,),
        in_specs=_fin_specs(),
        out_specs=pl.BlockSpec((NB, D), lambda i: (i, 0)),
        out_shape=jax.ShapeDtypeStruct((N, D), jnp.float32),
    )(acc, den, h, ss, sd, b[None, :], easum, We, a_e[None, :])


def _sc_edge_body(src_hbm, dst_hbm, ae_hbm, ss_hbm, sd_hbm, h_hbm,
                  acc_out, den_out,
                  ss_v, sd_v, denp_v, sidx_v, didx_v, ae_v, ex_v, rows_v,
                  acc_sh, sem):
    cid = lax.axis_index("c")
    sid = lax.axis_index("s")
    gid = cid * NS + sid
    ebase = gid * EPT
    nbase = sid * NPT

    pltpu.sync_copy(ss_hbm, ss_v)
    pltpu.sync_copy(sd_hbm, sd_v)

    def _zd(i, c):
        denp_v[pl.ds(i * 16, 16)] = jnp.zeros((16,), jnp.float32)
        return c
    lax.fori_loop(0, N // 16, _zd, 0)

    def _zr(i, c):
        r = i // (D // 16)
        k = i % (D // 16)
        rows_v[r, pl.ds(k * 16, 16)] = jnp.zeros((16,), jnp.float32)
        return c
    lax.fori_loop(0, CH * (D // 16), _zr, 0)

    nzfull = NPT // CH

    def _za(k, c):
        pltpu.sync_copy(rows_v, acc_sh.at[pl.ds(nbase + k * CH, CH)])
        return c
    lax.fori_loop(0, nzfull, _za, 0)
    rem = NPT - nzfull * CH
    if rem:
        pltpu.sync_copy(rows_v.at[pl.ds(0, rem)],
                        acc_sh.at[pl.ds(nbase + nzfull * CH, rem)])
    plsc.subcore_barrier()

    def _chunk(c, carry):
        base = ebase + c * CH
        pltpu.sync_copy(src_hbm.at[pl.ds(base, CH)], sidx_v)
        pltpu.sync_copy(dst_hbm.at[pl.ds(base, CH)], didx_v)
        pltpu.sync_copy(ae_hbm.at[pl.ds(base, CH)], ae_v)
        cp = pltpu.async_copy(h_hbm.at[sidx_v], rows_v, sem)

        def _s16(j, cc):
            sl = pl.ds(j * 16, 16)
            si = sidx_v[sl]
            di = didx_v[sl]
            a = (plsc.load_gather(ss_v, [si]) + plsc.load_gather(sd_v, [di])
                 + ae_v[sl])
            a = jnp.where(a >= 0.0, a, 0.2 * a)
            ex = jnp.exp(a)
            ex_v[sl] = ex
            plsc.addupdate_scatter(denp_v, [di], ex)
            return cc
        lax.fori_loop(0, CH // 16, _s16, 0)
        cp.wait()

        def _scale(j, cc):
            bex = plsc.load_gather(ex_v, [jnp.full((16,), j, jnp.int32)])
            for k in range(D // 16):
                sl = pl.ds(k * 16, 16)
                rows_v[j, sl] = rows_v[j, sl] * bex
            return cc
        lax.fori_loop(0, CH, _scale, 0)

        pltpu.sync_copy(rows_v, acc_sh.at[didx_v], add=True)
        return carry

    lax.fori_loop(0, NCHUNK, _chunk, 0)
    plsc.subcore_barrier()

    pltpu.sync_copy(denp_v, den_out.at[gid])
    pltpu.sync_copy(acc_sh.at[pl.ds(nbase, NPT)],
                    acc_out.at[cid, pl.ds(nbase, NPT)])


def _sc_edge_pass(src, dst, ae, ss, sd, h):
    mesh = plsc.VectorSubcoreMesh(core_axis_name="c", subcore_axis_name="s")
    f = pl.kernel(
        _sc_edge_body,
        out_type=[
            jax.ShapeDtypeStruct((NC, N, D), jnp.float32),
            jax.ShapeDtypeStruct((NT, N), jnp.float32),
        ],
        mesh=mesh,
        scratch_types=[
            pltpu.VMEM((N,), jnp.float32),
            pltpu.VMEM((N,), jnp.float32),
            pltpu.VMEM((N,), jnp.float32),
            pltpu.VMEM((CH,), jnp.int32),
            pltpu.VMEM((CH,), jnp.int32),
            pltpu.VMEM((CH,), jnp.float32),
            pltpu.VMEM((CH,), jnp.float32),
            pltpu.VMEM((CH, D), jnp.float32),
            pltpu.VMEM_SHARED((N, D), jnp.float32),
            pltpu.SemaphoreType.DMA,
        ],
    )
    return f(src, dst, ae, ss, sd, h)


def kernel(x, edge_index, edge_attr, batch,
           W1, a_src1, a_dst1, We1, a_e1, b1,
           W2, a_src2, a_dst2, We2, a_e2, b2,
           W3, a_src3, a_dst3, We3, a_e3, b3):
    del batch
    src = edge_index[0]
    dst = edge_index[1]

    ae1, ae2, ae3, easum = _edge_scalars(edge_attr, We1, a_e1, We2, a_e2,
                                         We3, a_e3)
    ae1 = ae1[:, 0]
    ae2 = ae2[:, 0]
    ae3 = ae3[:, 0]

    h1, ss1, sd1 = _prep(x, W1, a_src1, a_dst1)
    acc, den = _sc_edge_pass(src, dst, ae1, ss1[:, 0], sd1[:, 0], h1)
    h2, ss2, sd2 = _finalize_mid(acc, den, h1, ss1, sd1, b1, easum, We1, a_e1,
                                 W2, a_src2, a_dst2)
    acc, den = _sc_edge_pass(src, dst, ae2, ss2[:, 0], sd2[:, 0], h2)
    h3, ss3, sd3 = _finalize_mid(acc, den, h2, ss2, sd2, b2, easum, We2, a_e2,
                                 W3, a_src3, a_dst3)
    acc, den = _sc_edge_pass(src, dst, ae3, ss3[:, 0], sd3[:, 0], h3)
    return _finalize_last(acc, den, h3, ss3, sd3, b3, easum, We3, a_e3)


# speedup vs baseline: 15.5918x; 15.5918x over previous
import jax
import jax.numpy as jnp
from jax import lax
from jax.experimental import pallas as pl
from jax.experimental.pallas import tpu as pltpu
from jax.experimental.pallas import tpu_sc as plsc

N = 10000
E = 320000
D = 128
DE = 16

NB = 1024
EB = 4000

NC = 2
NS = 16
NT = NC * NS
EPT = E // NT
CH = 80
NCHUNK = EPT // CH
WSTRIDE = 624
WROWS = 640


def _prep_body(x_ref, w_ref, asrc_ref, adst_ref, h_ref, ss_ref, sd_ref):
    x = x_ref[...]
    nrm = jnp.sqrt(jnp.sum(x * x, axis=1, keepdims=True))
    nrm = jnp.where(nrm == 0.0, 1e-8, nrm)
    h = jnp.dot(x / nrm, w_ref[...], preferred_element_type=jnp.float32)
    h_ref[...] = h
    ss_ref[...] = jnp.sum(h * asrc_ref[...], axis=1, keepdims=True)
    sd_ref[...] = jnp.sum(h * adst_ref[...], axis=1, keepdims=True)


def _prep(x, W, a_src, a_dst):
    grid = (N + NB - 1) // NB
    return pl.pallas_call(
        _prep_body,
        grid=(grid,),
        in_specs=[
            pl.BlockSpec((NB, D), lambda i: (i, 0)),
            pl.BlockSpec((D, D), lambda i: (0, 0)),
            pl.BlockSpec((1, D), lambda i: (0, 0)),
            pl.BlockSpec((1, D), lambda i: (0, 0)),
        ],
        out_specs=[
            pl.BlockSpec((NB, D), lambda i: (i, 0)),
            pl.BlockSpec((NB, 1), lambda i: (i, 0)),
            pl.BlockSpec((NB, 1), lambda i: (i, 0)),
        ],
        out_shape=[
            jax.ShapeDtypeStruct((N, D), jnp.float32),
            jax.ShapeDtypeStruct((N, 1), jnp.float32),
            jax.ShapeDtypeStruct((N, 1), jnp.float32),
        ],
    )(x, W, a_src[None, :], a_dst[None, :])


def _ea_body(ea_ref, we1_ref, ae1_ref, we2_ref, ae2_ref, we3_ref, ae3_ref,
             o1_ref, o2_ref, o3_ref, easum_ref):
    ea = ea_ref[...]
    for we_ref, ae_ref, o_ref in ((we1_ref, ae1_ref, o1_ref),
                                  (we2_ref, ae2_ref, o2_ref),
                                  (we3_ref, ae3_ref, o3_ref)):
        v = jnp.sum(we_ref[...] * ae_ref[...], axis=1)
        o_ref[...] = jnp.sum(ea * v[None, :], axis=1, keepdims=True)

    @pl.when(pl.program_id(0) == 0)
    def _():
        easum_ref[...] = jnp.zeros_like(easum_ref)

    easum_ref[...] += jnp.sum(ea, axis=0, keepdims=True)


def _edge_scalars(edge_attr, We1, a_e1, We2, a_e2, We3, a_e3):
    grid = E // EB
    return pl.pallas_call(
        _ea_body,
        grid=(grid,),
        in_specs=[
            pl.BlockSpec((EB, DE), lambda i: (i, 0)),
            pl.BlockSpec((DE, D), lambda i: (0, 0)),
            pl.BlockSpec((1, D), lambda i: (0, 0)),
            pl.BlockSpec((DE, D), lambda i: (0, 0)),
            pl.BlockSpec((1, D), lambda i: (0, 0)),
            pl.BlockSpec((DE, D), lambda i: (0, 0)),
            pl.BlockSpec((1, D), lambda i: (0, 0)),
        ],
        out_specs=[
            pl.BlockSpec((EB, 1), lambda i: (i, 0)),
            pl.BlockSpec((EB, 1), lambda i: (i, 0)),
            pl.BlockSpec((EB, 1), lambda i: (i, 0)),
            pl.BlockSpec((1, DE), lambda i: (0, 0)),
        ],
        out_shape=[
            jax.ShapeDtypeStruct((E, 1), jnp.float32),
            jax.ShapeDtypeStruct((E, 1), jnp.float32),
            jax.ShapeDtypeStruct((E, 1), jnp.float32),
            jax.ShapeDtypeStruct((1, DE), jnp.float32),
        ],
    )(edge_attr, We1, a_e1[None, :], We2, a_e2[None, :], We3, a_e3[None, :])


def _gelu(o):
    return 0.5 * o * (1.0 + lax.erf(o * (2.0 ** -0.5)))


def _loop_ex(ss, sd, easum_ref, we_ref, ae_ref):
    v = jnp.sum(we_ref[...] * ae_ref[...], axis=1)
    loop_ae = jnp.sum(easum_ref[...] * v[None, :]) / E
    s = ss + sd + loop_ae
    s = jnp.where(s >= 0.0, s, 0.2 * s)
    return jnp.exp(s)


def _fin_mid_body(acc_ref, den_ref, h_ref, ss_ref, sd_ref, b_ref,
                  easum_ref, we_ref, ae_ref,
                  wn_ref, asrcn_ref, adstn_ref,
                  hn_ref, ssn_ref, sdn_ref):
    exl = _loop_ex(ss_ref[..., 0], sd_ref[..., 0], easum_ref, we_ref, ae_ref)
    h = h_ref[...]
    acc = acc_ref[0] + acc_ref[1] + exl[:, None] * h
    den = jnp.sum(den_ref[...], axis=0) + exl
    g = _gelu(acc / (den[:, None] + 1e-16) + b_ref[...])
    hn = jnp.dot(g, wn_ref[...], preferred_element_type=jnp.float32)
    hn_ref[...] = hn
    ssn_ref[...] = jnp.sum(hn * asrcn_ref[...], axis=1, keepdims=True)
    sdn_ref[...] = jnp.sum(hn * adstn_ref[...], axis=1, keepdims=True)


def _fin_last_body(acc_ref, den_ref, h_ref, ss_ref, sd_ref, b_ref,
                   easum_ref, we_ref, ae_ref, out_ref):
    exl = _loop_ex(ss_ref[..., 0], sd_ref[..., 0], easum_ref, we_ref, ae_ref)
    h = h_ref[...]
    acc = acc_ref[0] + acc_ref[1] + exl[:, None] * h
    den = jnp.sum(den_ref[...], axis=0) + exl
    out_ref[...] = _gelu(acc / (den[:, None] + 1e-16) + b_ref[...])


def _fin_specs():
    return [
        pl.BlockSpec((2, NB, D), lambda i: (0, i, 0)),
        pl.BlockSpec((NT, NB), lambda i: (0, i)),
        pl.BlockSpec((NB, D), lambda i: (i, 0)),
        pl.BlockSpec((NB, 1), lambda i: (i, 0)),
        pl.BlockSpec((NB, 1), lambda i: (i, 0)),
        pl.BlockSpec((1, D), lambda i: (0, 0)),
        pl.BlockSpec((1, DE), lambda i: (0, 0)),
        pl.BlockSpec((DE, D), lambda i: (0, 0)),
        pl.BlockSpec((1, D), lambda i: (0, 0)),
    ]


def _finalize_mid(acc, den, h, ss, sd, b, easum, We, a_e, Wn, asrcn, adstn):
    grid = (N + NB - 1) // NB
    return pl.pallas_call(
        _fin_mid_body,
        grid=(grid,),
        in_specs=_fin_specs() + [
            pl.BlockSpec((D, D), lambda i: (0, 0)),
            pl.BlockSpec((1, D), lambda i: (0, 0)),
            pl.BlockSpec((1, D), lambda i: (0, 0)),
        ],
        out_specs=[
            pl.BlockSpec((NB, D), lambda i: (i, 0)),
            pl.BlockSpec((NB, 1), lambda i: (i, 0)),
            pl.BlockSpec((NB, 1), lambda i: (i, 0)),
        ],
        out_shape=[
            jax.ShapeDtypeStruct((N, D), jnp.float32),
            jax.ShapeDtypeStruct((N, 1), jnp.float32),
            jax.ShapeDtypeStruct((N, 1), jnp.float32),
        ],
    )(acc, den, h, ss, sd, b[None, :], easum, We, a_e[None, :],
      Wn, asrcn[None, :], adstn[None, :])


def _finalize_last(acc, den, h, ss, sd, b, easum, We, a_e):
    grid = (N + NB - 1) // NB
    return pl.pallas_call(
        _fin_last_body,
        grid=(grid,),
        in_specs=_fin_specs(),
        out_specs=pl.BlockSpec((NB, D), lambda i: (i, 0)),
        out_shape=jax.ShapeDtypeStruct((N, D), jnp.float32),
    )(acc, den, h, ss, sd, b[None, :], easum, We, a_e[None, :])


def _sc_edge_body(src_hbm, dst_hbm, ae_hbm, ss_hbm, sd_hbm, h_hbm,
                  acc_out, den_out,
                  ss_v, sd_v, denp_v, sidx_v, didx_v, ae_v, ex_v, rows_v,
                  acc_sh, sem):
    cid = lax.axis_index("c")
    sid = lax.axis_index("s")
    gid = cid * NS + sid
    ebase = gid * EPT
    nbase = sid * WSTRIDE

    pltpu.sync_copy(ss_hbm, ss_v)
    pltpu.sync_copy(sd_hbm, sd_v)

    def _zd(i, c):
        denp_v[pl.ds(i * 16, 16)] = jnp.zeros((16,), jnp.float32)
        return c
    lax.fori_loop(0, N // 16, _zd, 0)

    def _zr(i, c):
        r = i // (D // 16)
        k = i % (D // 16)
        rows_v[r, pl.ds(k * 16, 16)] = jnp.zeros((16,), jnp.float32)
        return c
    lax.fori_loop(0, CH * (D // 16), _zr, 0)

    def _za(k, c):
        pltpu.sync_copy(rows_v, acc_sh.at[pl.ds(nbase + k * CH, CH)])
        return c
    lax.fori_loop(0, WROWS // CH, _za, 0)
    plsc.subcore_barrier()

    def _chunk(c, carry):
        base = ebase + c * CH
        pltpu.sync_copy(src_hbm.at[pl.ds(base, CH)], sidx_v)
        pltpu.sync_copy(dst_hbm.at[pl.ds(base, CH)], didx_v)
        pltpu.sync_copy(ae_hbm.at[pl.ds(base, CH)], ae_v)
        cp = pltpu.async_copy(h_hbm.at[sidx_v], rows_v, sem)

        def _s16(j, cc):
            sl = pl.ds(j * 16, 16)
            si = sidx_v[sl]
            di = didx_v[sl]
            a = (plsc.load_gather(ss_v, [si]) + plsc.load_gather(sd_v, [di])
                 + ae_v[sl])
            a = jnp.where(a >= 0.0, a, 0.2 * a)
            ex = jnp.exp(a)
            ex_v[sl] = ex
            plsc.addupdate_scatter(denp_v, [di], ex)
            return cc
        lax.fori_loop(0, CH // 16, _s16, 0)
        cp.wait()

        def _scale(j, cc):
            bex = plsc.load_gather(ex_v, [jnp.full((16,), j, jnp.int32)])
            for k in range(D // 16):
                sl = pl.ds(k * 16, 16)
                rows_v[j, sl] = rows_v[j, sl] * bex
            return cc
        lax.fori_loop(0, CH, _scale, 0)

        pltpu.sync_copy(rows_v, acc_sh.at[didx_v], add=True)
        return carry

    lax.fori_loop(0, NCHUNK, _chunk, 0)
    plsc.subcore_barrier()

    pltpu.sync_copy(denp_v, den_out.at[pl.ds(gid * N, N)])
    pltpu.sync_copy(acc_sh.at[pl.ds(nbase, WROWS)],
                    acc_out.at[cid, pl.ds(nbase, WROWS)])


def _sc_edge_pass(src, dst, ae, ss, sd, h):
    mesh = plsc.VectorSubcoreMesh(core_axis_name="c", subcore_axis_name="s")
    f = pl.kernel(
        _sc_edge_body,
        out_type=[
            jax.ShapeDtypeStruct((NC, N, D), jnp.float32),
            jax.ShapeDtypeStruct((NT * N,), jnp.float32),
        ],
        mesh=mesh,
        compiler_params=pltpu.CompilerParams(needs_layout_passes=False),
        scratch_types=[
            pltpu.VMEM((N,), jnp.float32),
            pltpu.VMEM((N,), jnp.float32),
            pltpu.VMEM((N,), jnp.float32),
            pltpu.VMEM((CH,), jnp.int32),
            pltpu.VMEM((CH,), jnp.int32),
            pltpu.VMEM((CH,), jnp.float32),
            pltpu.VMEM((CH,), jnp.float32),
            pltpu.VMEM((CH, D), jnp.float32),
            pltpu.VMEM_SHARED((N, D), jnp.float32),
            pltpu.SemaphoreType.DMA,
        ],
    )
    acc, den = f(src, dst, ae, ss, sd, h)
    return acc, den.reshape(NT, N)


def kernel(x, edge_index, edge_attr, batch,
           W1, a_src1, a_dst1, We1, a_e1, b1,
           W2, a_src2, a_dst2, We2, a_e2, b2,
           W3, a_src3, a_dst3, We3, a_e3, b3):
    del batch
    src = edge_index[0]
    dst = edge_index[1]

    ae1, ae2, ae3, easum = _edge_scalars(edge_attr, We1, a_e1, We2, a_e2,
                                         We3, a_e3)
    ae1 = ae1[:, 0]
    ae2 = ae2[:, 0]
    ae3 = ae3[:, 0]

    h1, ss1, sd1 = _prep(x, W1, a_src1, a_dst1)
    acc, den = _sc_edge_pass(src, dst, ae1, ss1[:, 0], sd1[:, 0], h1)
    h2, ss2, sd2 = _finalize_mid(acc, den, h1, ss1, sd1, b1, easum, We1, a_e1,
                                 W2, a_src2, a_dst2)
    acc, den = _sc_edge_pass(src, dst, ae2, ss2[:, 0], sd2[:, 0], h2)
    h3, ss3, sd3 = _finalize_mid(acc, den, h2, ss2, sd2, b2, easum, We2, a_e2,
                                 W3, a_src3, a_dst3)
    acc, den = _sc_edge_pass(src, dst, ae3, ss3[:, 0], sd3[:, 0], h3)
    return _finalize_last(acc, den, h3, ss3, sd3, b3, easum, We3, a_e3)
